# Initial kernel scaffold; baseline (speedup 1.0000x reference)
import jax
import jax.numpy as jnp
from jax.experimental import pallas as pl


def kernel(m_ji, nbr_list, angle_list, e_rbf, a_sbf, kj_idx, W_nbr, b_nbr, W_erbf, W_asbf, final_w):
    raise NotImplementedError("write your pallas kernel here")


# speedup vs baseline: 5.1421x; 5.1421x over previous
import functools

import jax
import jax.numpy as jnp
from jax import lax
from jax.experimental import pallas as pl
from jax.experimental.pallas import tpu as pltpu
from jax.experimental.pallas import tpu_sc as plsc

NB = 8
NW = 32
NSUB = 16
BATCH = 128


def _ta_body(a_ref, w_ref, o_ref):
    o_ref[...] = jnp.dot(a_ref[...], w_ref[...],
                         preferred_element_type=jnp.float32)


def _transf_a(a_sbf, W_asbf, block_a=2000):
    A, AD = a_sbf.shape
    grid = A // block_a
    return pl.pallas_call(
        _ta_body,
        grid=(grid,),
        in_specs=[
            pl.BlockSpec((block_a, AD), lambda i: (i, 0)),
            pl.BlockSpec((AD, NB), lambda i: (0, 0)),
        ],
        out_specs=pl.BlockSpec((block_a, NB), lambda i: (i, 0)),
        out_shape=jax.ShapeDtypeStruct((A, NB), jnp.float32),
    )(a_sbf, W_asbf)


CH = 8


def _make_segsum(E, n_batch):
    ept = E // NSUB
    n_chunk = n_batch // CH
    rows_per_chunk = CH * BATCH
    mesh = plsc.VectorSubcoreMesh(core_axis_name="c", subcore_axis_name="s")

    @functools.partial(
        pl.kernel,
        mesh=mesh,
        compiler_params=pltpu.CompilerParams(use_tc_tiling_on_sc=False),
        out_type=jax.ShapeDtypeStruct((2, E, NB), jnp.float32),
        scratch_types=[
            pltpu.VMEM_SHARED((E, NB), jnp.float32),
            pltpu.VMEM((n_batch, BATCH), jnp.int32),
            pltpu.VMEM((rows_per_chunk, NB), jnp.float32),
        ],
    )
    def segsum(kj_ref, val_ref, zero_ref, out_ref, acc, idx_v, val_c):
        c = lax.axis_index("c")
        s = lax.axis_index("s")
        wid = c * NSUB + s
        pltpu.sync_copy(zero_ref.at[pl.ds(s * ept, ept)],
                        acc.at[pl.ds(s * ept, ept)])
        plsc.subcore_barrier()
        pltpu.sync_copy(kj_ref.at[wid], idx_v)

        def chunk_body(k, carry):
            pltpu.sync_copy(
                val_ref.at[wid, pl.ds(k * rows_per_chunk, rows_per_chunk)],
                val_c)
            def b_iter(b, c2):
                pltpu.sync_copy(val_c.at[pl.ds(b * BATCH, BATCH)],
                                acc.at[idx_v.at[k * CH + b]], add=True)
                return c2
            lax.fori_loop(0, CH, b_iter, 0)
            return carry
        lax.fori_loop(0, n_chunk, chunk_body, 0)
        plsc.subcore_barrier()
        pltpu.sync_copy(acc.at[pl.ds(s * ept, ept)],
                        out_ref.at[c, pl.ds(s * ept, ept)])

    return segsum


def _segment_sum_sc(t_a, kj_idx, E):
    A = t_a.shape[0]
    rows_per_chunk = CH * BATCH
    per_tile = -(-A // (NW * rows_per_chunk)) * rows_per_chunk
    n_batch = per_tile // BATCH
    a_pad = per_tile * NW
    kj32 = kj_idx.astype(jnp.int32)
    kj_p = jnp.pad(kj32, (0, a_pad - A)).reshape(NW, n_batch, BATCH)
    val_p = jnp.pad(t_a, ((0, a_pad - A), (0, 0))).reshape(NW, per_tile, NB)
    zeros = jnp.zeros((E, NB), jnp.float32)
    return _make_segsum(E, n_batch)(kj_p, val_p, zeros)


def _final_body(m_ref, e_ref, s_ref, wn_ref, bn_ref, we_ref, f2_ref, o_ref):
    pre = jnp.dot(m_ref[...], wn_ref[...],
                  preferred_element_type=jnp.float32) + bn_ref[...]
    tn = pre * jax.nn.sigmoid(pre)
    te = jnp.dot(e_ref[...], we_ref[...], preferred_element_type=jnp.float32)
    me = tn * te
    s = s_ref[0] + s_ref[1]
    acc = jnp.dot(me * s[:, 0:1], f2_ref[0], preferred_element_type=jnp.float32)
    for j in range(1, NB):
        acc = acc + jnp.dot(me * s[:, j:j + 1], f2_ref[j],
                            preferred_element_type=jnp.float32)
    o_ref[...] = acc


def _final(m_ji, e_rbf, s2, W_nbr, b_nbr, W_erbf, F2, block_e=1000):
    E, D = m_ji.shape
    NR = e_rbf.shape[1]
    grid = E // block_e
    return pl.pallas_call(
        _final_body,
        grid=(grid,),
        in_specs=[
            pl.BlockSpec((block_e, D), lambda i: (i, 0)),
            pl.BlockSpec((block_e, NR), lambda i: (i, 0)),
            pl.BlockSpec((2, block_e, NB), lambda i: (0, i, 0)),
            pl.BlockSpec((D, D), lambda i: (0, 0)),
            pl.BlockSpec((1, D), lambda i: (0, 0)),
            pl.BlockSpec((NR, D), lambda i: (0, 0)),
            pl.BlockSpec((NB, D, D), lambda i: (0, 0, 0)),
        ],
        out_specs=pl.BlockSpec((block_e, D), lambda i: (i, 0)),
        out_shape=jax.ShapeDtypeStruct((E, D), jnp.float32),
    )(m_ji, e_rbf, s2, W_nbr, b_nbr, W_erbf, F2)


def kernel(m_ji, nbr_list, angle_list, e_rbf, a_sbf, kj_idx,
           W_nbr, b_nbr, W_erbf, W_asbf, final_w):
    E, D = m_ji.shape
    t_a = _transf_a(a_sbf, W_asbf)
    s2 = _segment_sum_sc(t_a, kj_idx, E)
    F2 = jnp.transpose(final_w, (1, 2, 0))
    return _final(m_ji, e_rbf, s2, W_nbr,
                  b_nbr.reshape(1, D), W_erbf, F2)
